# baseline (speedup 1.0000x reference)
import jax
import jax.numpy as jnp
from jax import lax
from jax.experimental import pallas as pl
from jax.experimental.pallas import tpu as pltpu
from jax.experimental.pallas import tpu_sc as plsc

NC = 2
NS = 16
NW = NC * NS
L = 16

_B = 8192
_T = 16
_ML = 543
_WIN = 64
_NSPLIT = 1
_STARTS = tuple(range(0, _ML - L + 1, L)) + ((_ML - L),)
_HALVES = (_STARTS[:17], _STARTS[17:])
_CH = 4


def _chunks(seq, n):
    return [seq[i:i + n] for i in range(0, len(seq), n)]


def _make_body(bpw):
    nwin = bpw // _WIN

    def _sc_body(inp_hbm, tid_hbm, off_hbm, disc_hbm, mask_hbm, out_hbm,
                 tid_v, inp_v, off_v, disc_v, mask_v, out_v, ctrs, slots,
                 sem):
        wid = lax.axis_index("s") * NC + lax.axis_index("c")
        base = wid * bpw

        cps = [
            pltpu.async_copy(tid_hbm.at[pl.ds(base, bpw)], tid_v, sem),
            pltpu.async_copy(inp_hbm.at[pl.ds(base, bpw)],
                             inp_v.at[pl.ds(0, bpw)], sem),
            pltpu.async_copy(off_hbm, off_v, sem),
            pltpu.async_copy(disc_hbm, disc_v, sem),
            pltpu.async_copy(mask_hbm, mask_v, sem),
        ]
        for cp in cps:
            cp.wait()

        def fuse_row(t, _):
            for blks in _chunks(_STARTS, _CH):
                ds_ = [disc_v[t, pl.ds(st, L)] for st in blks]
                ms = [mask_v[t, pl.ds(st, L)] for st in blks]
                os_ = [off_v[t, pl.ds(st, L)] for st in blks]
                as_ = [d * m for d, m in zip(ds_, ms)]
                cs = [a * o for a, o in zip(as_, os_)]
                for st, a in zip(blks, as_):
                    disc_v[t, pl.ds(st, L)] = a
                for st, c in zip(blks, cs):
                    off_v[t, pl.ds(st, L)] = c
            return 0
        lax.fori_loop(0, _T, fuse_row, 0)

        for w in range(nwin):
            wbase = w * _WIN
            b = w % 2

            if w >= 2:
                pltpu.make_async_copy(
                    out_hbm.at[pl.ds(base, _WIN)], out_v.at[0], sem).wait()

            for t in range(_T):
                ctrs[t] = 0
            for gg in range(_WIN // L):
                tid16 = tid_v[pl.ds(wbase + gg * L, L)]
                for k in range(L):
                    t = tid16[k]
                    cnt = ctrs[t]
                    slots[t * _WIN + cnt] = gg * L + k
                    ctrs[t] = cnt + 1

            def task_body(t, _):
                cnt = ctrs[t]
                for half in _HALVES:
                    areg = [disc_v[t, pl.ds(st, L)] for st in half]
                    creg = [off_v[t, pl.ds(st, L)] for st in half]

                    def emit_row(j):
                        slot = slots[t * _WIN + j]
                        s = inp_v[pl.ds(wbase + slot, L)][0]
                        return slot, s

                    def pair_body(j, _):
                        slot0, s0 = emit_row(2 * j)
                        slot1, s1 = emit_row(2 * j + 1)
                        for qs in _chunks(tuple(range(17)), _CH):
                            o0 = [areg[q] * s0 + creg[q] for q in qs]
                            o1 = [areg[q] * s1 + creg[q] for q in qs]
                            for q, o in zip(qs, o0):
                                out_v[b, slot0, pl.ds(half[q], L)] = o
                            for q, o in zip(qs, o1):
                                out_v[b, slot1, pl.ds(half[q], L)] = o
                        return 0
                    lax.fori_loop(0, lax.div(cnt, 2), pair_body, 0)

                    @pl.when(lax.rem(cnt, 2) == 1)
                    def _tail():
                        slot, s = emit_row(cnt - 1)
                        for qs in _chunks(tuple(range(17)), _CH):
                            outs = [areg[q] * s + creg[q] for q in qs]
                            for q, o in zip(qs, outs):
                                out_v[b, slot, pl.ds(half[q], L)] = o
                return 0
            lax.fori_loop(0, _T, task_body, 0)

            pltpu.async_copy(out_v.at[b],
                             out_hbm.at[pl.ds(base + wbase, _WIN)], sem)

        for _ in range(min(2, nwin)):
            pltpu.make_async_copy(
                out_hbm.at[pl.ds(base, _WIN)], out_v.at[0], sem).wait()

    return _sc_body


def _make_kernel(nb):
    bpw = nb // NW
    return pl.kernel(
        _make_body(bpw),
        out_type=jax.ShapeDtypeStruct((nb, _ML), jnp.float32),
        mesh=plsc.VectorSubcoreMesh(core_axis_name="c", subcore_axis_name="s"),
        compiler_params=pltpu.CompilerParams(needs_layout_passes=False),
        scratch_types=[
            pltpu.VMEM((bpw,), jnp.int32),
            pltpu.VMEM((bpw + L,), jnp.float32),
            pltpu.VMEM((_T, _ML), jnp.float32),
            pltpu.VMEM((_T, _ML), jnp.float32),
            pltpu.VMEM((_T, _ML), jnp.float32),
            pltpu.VMEM((2, _WIN, _ML), jnp.float32),
            pltpu.SMEM((_T,), jnp.int32),
            pltpu.SMEM((_T * _WIN,), jnp.int32),
            pltpu.SemaphoreType.DMA,
        ],
    )


@jax.jit
def _sc_affine(inp1, task_ids, offsets, discrimination, mask):
    nb = _B // _NSPLIT
    kfn = _make_kernel(nb)
    outs = []
    for i in range(_NSPLIT):
        sl = slice(i * nb, (i + 1) * nb)
        outs.append(kfn(inp1[sl], task_ids[sl], offsets, discrimination,
                        mask))
    return jnp.concatenate(outs, axis=0)


def kernel(inp, task_ids, offsets, discrimination, mask):
    return _sc_affine(inp.reshape(-1), task_ids, offsets, discrimination,
                      mask)


# speedup vs baseline: 1.0355x; 1.0355x over previous
import jax
import jax.numpy as jnp
from jax import lax
from jax.experimental import pallas as pl
from jax.experimental.pallas import tpu as pltpu
from jax.experimental.pallas import tpu_sc as plsc

NC = 2
NS = 16
NW = NC * NS
L = 16

_B = 8192
_T = 16
_ML = 543
_WIN = 64
_NSPLIT = 1
_STARTS = tuple(range(0, _ML - L + 1, L)) + ((_ML - L),)
_HALVES = (_STARTS[:17], _STARTS[17:])
_CH = 4


def _chunks(seq, n):
    return [seq[i:i + n] for i in range(0, len(seq), n)]


def _make_body(bpw):
    nwin = bpw // _WIN

    def _sc_body(inp_hbm, tid_hbm, off_hbm, disc_hbm, mask_hbm, out_hbm,
                 tid_v, inp_v, off_v, disc_v, mask_v, out_v, ctrs, slots,
                 sem):
        wid = lax.axis_index("s") * NC + lax.axis_index("c")
        base = wid * bpw

        cps = [
            pltpu.async_copy(tid_hbm.at[pl.ds(base, bpw)], tid_v, sem),
            pltpu.async_copy(inp_hbm.at[pl.ds(base, bpw)],
                             inp_v.at[pl.ds(0, bpw)], sem),
            pltpu.async_copy(off_hbm, off_v, sem),
            pltpu.async_copy(disc_hbm, disc_v, sem),
            pltpu.async_copy(mask_hbm, mask_v, sem),
        ]
        for cp in cps:
            cp.wait()

        def fuse_row(t, _):
            for blks in _chunks(_STARTS, _CH):
                ds_ = [disc_v[t, pl.ds(st, L)] for st in blks]
                ms = [mask_v[t, pl.ds(st, L)] for st in blks]
                os_ = [off_v[t, pl.ds(st, L)] for st in blks]
                as_ = [d * m for d, m in zip(ds_, ms)]
                cs = [a * o for a, o in zip(as_, os_)]
                for st, a in zip(blks, as_):
                    disc_v[t, pl.ds(st, L)] = a
                for st, c in zip(blks, cs):
                    off_v[t, pl.ds(st, L)] = c
            return 0
        lax.fori_loop(0, _T, fuse_row, 0)

        for w in range(nwin):
            wbase = w * _WIN
            b = w % 2

            if w >= 2:
                pltpu.make_async_copy(
                    out_hbm.at[pl.ds(base, _WIN)], out_v.at[0], sem).wait()

            for t in range(_T):
                ctrs[t] = 0
            for gg in range(_WIN // L):
                tid16 = tid_v[pl.ds(wbase + gg * L, L)]
                for k in range(L):
                    t = tid16[k]
                    cnt = ctrs[t]
                    slots[t * _WIN + cnt] = gg * L + k
                    ctrs[t] = cnt + 1

            def task_body(t, _):
                cnt = ctrs[t]
                for half in _HALVES:
                    areg = [disc_v[t, pl.ds(st, L)] for st in half]
                    creg = [off_v[t, pl.ds(st, L)] for st in half]

                    def row_body(j, _):
                        slot = slots[t * _WIN + j]
                        sv = inp_v[pl.ds(wbase + slot, L)]
                        s = sv[0]
                        for qs in _chunks(tuple(range(17)), _CH):
                            outs = [areg[q] * s + creg[q] for q in qs]
                            for q, o in zip(qs, outs):
                                out_v[b, slot, pl.ds(half[q], L)] = o
                        return 0
                    lax.fori_loop(0, cnt, row_body, 0)
                return 0
            lax.fori_loop(0, _T, task_body, 0)

            pltpu.async_copy(out_v.at[b],
                             out_hbm.at[pl.ds(base + wbase, _WIN)], sem)

        for _ in range(min(2, nwin)):
            pltpu.make_async_copy(
                out_hbm.at[pl.ds(base, _WIN)], out_v.at[0], sem).wait()

    return _sc_body


def _make_kernel(nb):
    bpw = nb // NW
    return pl.kernel(
        _make_body(bpw),
        out_type=jax.ShapeDtypeStruct((nb, _ML), jnp.float32),
        mesh=plsc.VectorSubcoreMesh(core_axis_name="c", subcore_axis_name="s"),
        compiler_params=pltpu.CompilerParams(needs_layout_passes=False),
        scratch_types=[
            pltpu.VMEM((bpw,), jnp.int32),
            pltpu.VMEM((bpw + L,), jnp.float32),
            pltpu.VMEM((_T, _ML), jnp.float32),
            pltpu.VMEM((_T, _ML), jnp.float32),
            pltpu.VMEM((_T, _ML), jnp.float32),
            pltpu.VMEM((2, _WIN, _ML), jnp.float32),
            pltpu.SMEM((_T,), jnp.int32),
            pltpu.SMEM((_T * _WIN,), jnp.int32),
            pltpu.SemaphoreType.DMA,
        ],
    )


@jax.jit
def _sc_affine(inp1, task_ids, offsets, discrimination, mask):
    nb = _B // _NSPLIT
    kfn = _make_kernel(nb)
    outs = []
    for i in range(_NSPLIT):
        sl = slice(i * nb, (i + 1) * nb)
        outs.append(kfn(inp1[sl], task_ids[sl], offsets, discrimination,
                        mask))
    return jnp.concatenate(outs, axis=0)


def kernel(inp, task_ids, offsets, discrimination, mask):
    return _sc_affine(inp.reshape(-1), task_ids, offsets, discrimination,
                      mask)
